# Initial kernel scaffold; baseline (speedup 1.0000x reference)
import jax
import jax.numpy as jnp
from jax.experimental import pallas as pl


def kernel(x, Wr, br, W1, W2, W3, gamma, beta):
    raise NotImplementedError("write your pallas kernel here")


# speedup vs baseline: 1.0352x; 1.0352x over previous
import functools

import jax
import jax.numpy as jnp
from jax.experimental import pallas as pl
from jax.experimental.pallas import tpu as pltpu

_B, _N, _D = 2, 2048, 768
_E = 8
_F = 3072
_EPS = 1e-5

_TM = 512
_FT = 768
_NB = (_B * _N) // _TM
_NF = _F // _FT


def _router_body(x_ref, wr_ref, br_ref, cwt_ref):
    x = x_ref[...]
    logits = jnp.dot(x, wr_ref[...], preferred_element_type=jnp.float32)
    logits = logits + br_ref[...][None, :]
    m = jnp.max(logits, axis=-1, keepdims=True)
    ex = jnp.exp(logits - m)
    probs = ex / jnp.sum(ex, axis=-1, keepdims=True)
    T = probs.shape[0]
    e_ids = jax.lax.broadcasted_iota(jnp.int32, (T, _E), 1)
    m1 = jnp.max(probs, axis=-1, keepdims=True)
    idx1 = jnp.min(jnp.where(probs == m1, e_ids, _E), axis=-1, keepdims=True)
    probs2 = jnp.where(e_ids == idx1, -1.0, probs)
    m2 = jnp.max(probs2, axis=-1, keepdims=True)
    idx2 = jnp.min(jnp.where(probs2 == m2, e_ids, _E), axis=-1, keepdims=True)
    s = m1 + m2
    w1 = m1 / s
    w2 = m2 / s
    cw = jnp.where(e_ids == idx1, w1, 0.0) + jnp.where(e_ids == idx2, w2, 0.0)
    cwt_ref[...] = cw.T


def _ffn_body(x_ref, w1_ref, w3_ref, w2_ref, g_ref, b_ref, cwt_ref,
              out_ref, y_acc):
    f = pl.program_id(2)
    e = pl.program_id(1)
    x = x_ref[...]
    a = jnp.dot(x, w1_ref[0], preferred_element_type=jnp.float32)
    g = jnp.dot(x, w3_ref[0], preferred_element_type=jnp.float32)
    h = (a * jax.lax.logistic(a)) * g
    contrib = jnp.dot(h, w2_ref[0], preferred_element_type=jnp.float32)

    @pl.when(f == 0)
    def _():
        y_acc[...] = contrib

    @pl.when(f > 0)
    def _():
        y_acc[...] += contrib

    @pl.when(f == _NF - 1)
    def _():
        y = y_acc[...]
        mu = jnp.mean(y, axis=-1, keepdims=True)
        d = y - mu
        var = jnp.mean(d * d, axis=-1, keepdims=True)
        yn = d * jax.lax.rsqrt(var + _EPS) * g_ref[0] + b_ref[0]
        wk = cwt_ref[0, 0][:, None]
        contrib_out = wk * yn

        @pl.when(e == 0)
        def _():
            out_ref[...] = contrib_out

        @pl.when(e > 0)
        def _():
            out_ref[...] += contrib_out


@jax.jit
def kernel(x, Wr, br, W1, W2, W3, gamma, beta):
    Bb, Nn, Dd = x.shape
    T = Bb * Nn
    xf = x.reshape(T, Dd)

    cwt = pl.pallas_call(
        _router_body,
        out_shape=jax.ShapeDtypeStruct((_E, T), jnp.float32),
    )(xf, Wr, br)

    out = pl.pallas_call(
        _ffn_body,
        grid=(_NB, _E, _NF),
        in_specs=[
            pl.BlockSpec((_TM, _D), lambda b, e, f: (b, 0)),
            pl.BlockSpec((1, _D, _FT), lambda b, e, f: (e, 0, f)),
            pl.BlockSpec((1, _D, _FT), lambda b, e, f: (e, 0, f)),
            pl.BlockSpec((1, _FT, _D), lambda b, e, f: (e, f, 0)),
            pl.BlockSpec((1, 1, _D), lambda b, e, f: (e, 0, 0)),
            pl.BlockSpec((1, 1, _D), lambda b, e, f: (e, 0, 0)),
            pl.BlockSpec((1, 1, _TM), lambda b, e, f: (e, 0, b)),
        ],
        out_specs=pl.BlockSpec((_TM, _D), lambda b, e, f: (b, 0)),
        out_shape=jax.ShapeDtypeStruct((T, _D), jnp.float32),
        scratch_shapes=[pltpu.VMEM((_TM, _D), jnp.float32)],
    )(xf, W1, W3, W2, gamma.reshape(_E, 1, _D), beta.reshape(_E, 1, _D),
      cwt.reshape(_E, 1, T))

    return out.reshape(Bb, Nn, Dd)


# speedup vs baseline: 2.2496x; 2.1732x over previous
import functools

import jax
import jax.numpy as jnp
from jax import lax
from jax.experimental import pallas as pl
from jax.experimental.pallas import tpu as pltpu
from jax.experimental.pallas import tpu_sc as plsc

_B, _N, _D = 2, 2048, 768
_E = 8
_F = 3072
_EPS = 1e-5

_T = _B * _N
_S = 2 * _T
_TM = 512
_NT = _S // _TM + _E - 1
_SLOTS = _NT * _TM
_FT = 768
_NF = _F // _FT

_EPT = _S // 16


def _router_body(x_ref, wr_ref, br_ref, sel_ref, w_ref):
    x = x_ref[...]
    logits = jnp.dot(x, wr_ref[...], preferred_element_type=jnp.float32)
    logits = logits + br_ref[...][None, :]
    m = jnp.max(logits, axis=-1, keepdims=True)
    ex = jnp.exp(logits - m)
    probs = ex / jnp.sum(ex, axis=-1, keepdims=True)
    e_ids = lax.broadcasted_iota(jnp.int32, (_T, _E), 1)
    m1 = jnp.max(probs, axis=-1, keepdims=True)
    idx1 = jnp.min(jnp.where(probs == m1, e_ids, _E), axis=-1, keepdims=True)
    probs2 = jnp.where(e_ids == idx1, -1.0, probs)
    m2 = jnp.max(probs2, axis=-1, keepdims=True)
    idx2 = jnp.min(jnp.where(probs2 == m2, e_ids, _E), axis=-1, keepdims=True)
    s = m1 + m2
    k_ids = lax.broadcasted_iota(jnp.int32, (_T, 2), 1)
    sel_ref[...] = jnp.where(k_ids == 0, idx1, idx2)
    w_ref[...] = jnp.where(k_ids == 0, m1 / s, m2 / s)


def _splat(s, dtype=jnp.int32):
    return lax.broadcast_in_dim(jnp.asarray(s, dtype), (16,), ())


def _meta_body(sel_h, pos_h, te_h, nv_h,
               sel_v, rank_v, slots_v, sb_v, cntloc_v, cntflat_v,
               tot_r, gs_r, te_v, nv_v, cnt_sh):
    cid = lax.axis_index("c")
    sid = lax.axis_index("s")
    iota = lax.iota(jnp.int32, 16)

    @pl.when(cid == 0)
    def _stage_a():
        pltpu.sync_copy(sel_h.at[pl.ds(sid * 4, 4)], sel_v)
        cnt = [jnp.zeros((16,), jnp.int32)] * _E
        for q in range(4):
            for r8 in range(8):
                v = sel_v[q, pl.ds(r8 * 16, 16)]
                rk = jnp.zeros((16,), jnp.int32)
                for e in range(_E):
                    mres = v == _splat(e)
                    inc = jnp.where(mres, 1, 0).astype(jnp.int32)
                    c = plsc.cumsum(inc)
                    tot = _splat(jnp.sum(inc))
                    rk = jnp.where(mres, c - 1 + cnt[e], rk)
                    cnt[e] = cnt[e] + tot
                rank_v[q, pl.ds(r8 * 16, 16)] = rk
        acc = jnp.zeros((16,), jnp.int32)
        for e in range(_E):
            acc = acc + jnp.where(iota == _splat(e), cnt[e],
                                  jnp.zeros((16,), jnp.int32))
        cntloc_v[...] = acc
        pltpu.sync_copy(cntloc_v, cnt_sh.at[pl.ds(sid * 16, 16)])

    plsc.subcore_barrier()

    @pl.when(cid == 0)
    def _stage_b():
        pltpu.sync_copy(cnt_sh, cntflat_v)
        tot_v = jnp.zeros((16,), jnp.int32)
        base_v = jnp.zeros((16,), jnp.int32)
        sid_v = _splat(sid)
        for t in range(16):
            row = cntflat_v[pl.ds(t * 16, 16)]
            tot_v = tot_v + row
            base_v = base_v + jnp.where(_splat(t) < sid_v, row,
                                        jnp.zeros((16,), jnp.int32))
        pc = jnp.bitwise_and(tot_v + _splat(_TM - 1), _splat(~(_TM - 1)))
        gs = plsc.cumsum(pc) - pc
        sb_v[...] = gs + base_v
        for q in range(4):
            for r8 in range(8):
                v = sel_v[q, pl.ds(r8 * 16, 16)]
                rk = rank_v[q, pl.ds(r8 * 16, 16)]
                sb = plsc.load_gather(sb_v, [v])
                slots_v[q, pl.ds(r8 * 16, 16)] = sb + rk
        pltpu.sync_copy(slots_v, pos_h.at[pl.ds(sid * 4, 4)])

        @pl.when(sid == 0)
        def _te():
            tot_r[...] = tot_v
            gs_r[...] = gs
            ones = _splat(1)
            zeros = jnp.zeros((16,), jnp.int32)
            for j in range(3):
                pv = (iota + _splat(j * 16)) * _splat(_TM)
                cntv = jnp.zeros((16,), jnp.int32)
                for e in range(_E):
                    gse = _splat(jnp.sum(jnp.where(iota == _splat(e), gs, zeros)))
                    cntv = cntv + jnp.where(pv >= gse, ones, zeros)
                te = jnp.minimum(jnp.maximum(cntv - ones, zeros),
                                 _splat(_E - 1))
                te_v[pl.ds(j * 16, 16)] = te
                cg = plsc.load_gather(tot_r, [te])
                gsg = plsc.load_gather(gs_r, [te])
                nv = jnp.minimum(cg - (pv - gsg), _splat(_TM))
                nv_v[pl.ds(j * 16, 16)] = jnp.maximum(nv, zeros)
            pltpu.sync_copy(te_v, te_h)
            pltpu.sync_copy(nv_v, nv_h)


def _dispatch_body(x_h, pos_h, xs_h, p0_v, p1_v, rows_v, sem):
    cid = lax.axis_index("c")
    sid = lax.axis_index("s")
    wid = sid * 2 + cid
    tok_base = wid * 128
    pltpu.sync_copy(pos_h.at[wid], p0_v)
    pltpu.sync_copy(pos_h.at[32 + wid], p1_v)
    pltpu.sync_copy(x_h.at[pl.ds(tok_base, 128)], rows_v)
    pltpu.async_copy(rows_v, xs_h.at[p0_v], sem).wait()
    pltpu.async_copy(rows_v, xs_h.at[p1_v], sem).wait()


def _ffn_body(te_ref, nv_ref, xs_ref, w1_ref, w3_ref, w2_ref, g_ref, b_ref,
              out_ref, y_acc):
    t = pl.program_id(0)
    f = pl.program_id(1)
    valid = nv_ref[t] > 0

    @pl.when(valid)
    def _():
        x = xs_ref[...]
        a = jnp.dot(x, w1_ref[0], preferred_element_type=jnp.float32)
        g = jnp.dot(x, w3_ref[0], preferred_element_type=jnp.float32)
        h = (a * lax.logistic(a)) * g
        contrib = jnp.dot(h, w2_ref[0], preferred_element_type=jnp.float32)

        @pl.when(f == 0)
        def _():
            y_acc[...] = contrib

        @pl.when(f > 0)
        def _():
            y_acc[...] += contrib

        @pl.when(f == _NF - 1)
        def _():
            y = y_acc[...]
            mu = jnp.mean(y, axis=-1, keepdims=True)
            d = y - mu
            var = jnp.mean(d * d, axis=-1, keepdims=True)
            out_ref[...] = d * lax.rsqrt(var + _EPS) * g_ref[0] + b_ref[0]


def _combine_body(ybuf_h, pos_h, w_h, out_h,
                  p0_v, p1_v, w0_v, w1_v, rows0_v, rows1_v, acc_v, sem):
    cid = lax.axis_index("c")
    sid = lax.axis_index("s")
    wid = sid * 2 + cid
    tok_base = wid * 128
    pltpu.sync_copy(pos_h.at[wid], p0_v)
    pltpu.sync_copy(pos_h.at[32 + wid], p1_v)
    pltpu.sync_copy(w_h.at[wid], w0_v)
    pltpu.sync_copy(w_h.at[32 + wid], w1_v)
    for q in range(4):
        pltpu.async_copy(ybuf_h.at[p0_v.at[pl.ds(q * 32, 32)]], rows0_v,
                         sem).wait()
        pltpu.async_copy(ybuf_h.at[p1_v.at[pl.ds(q * 32, 32)]], rows1_v,
                         sem).wait()

        def add_tok(i, carry):
            w0 = plsc.load_gather(w0_v, [_splat(q * 32) + _splat(i)])
            w1 = plsc.load_gather(w1_v, [_splat(q * 32) + _splat(i)])
            for c in range(_D // 16):
                r0 = rows0_v[i, pl.ds(c * 16, 16)]
                r1 = rows1_v[i, pl.ds(c * 16, 16)]
                acc_v[i, pl.ds(c * 16, 16)] = w0 * r0 + w1 * r1
            return carry

        lax.fori_loop(0, 32, add_tok, 0)
        pltpu.sync_copy(acc_v, out_h.at[pl.ds(tok_base + q * 32, 32)])


_MESH = plsc.VectorSubcoreMesh(core_axis_name="c", subcore_axis_name="s")
_SC_PARAMS = pltpu.CompilerParams(needs_layout_passes=False)


def _meta_kernel():
    return functools.partial(
        pl.kernel,
        mesh=_MESH,
        compiler_params=_SC_PARAMS,
        out_type=[jax.ShapeDtypeStruct((_S // 128, 128), jnp.int32),
                  jax.ShapeDtypeStruct((48,), jnp.int32),
                  jax.ShapeDtypeStruct((48,), jnp.int32)],
        scratch_types=[
            pltpu.VMEM((4, 128), jnp.int32),
            pltpu.VMEM((4, 128), jnp.int32),
            pltpu.VMEM((4, 128), jnp.int32),
            pltpu.VMEM((16,), jnp.int32),
            pltpu.VMEM((16,), jnp.int32),
            pltpu.VMEM((256,), jnp.int32),
            pltpu.VMEM((16,), jnp.int32),
            pltpu.VMEM((16,), jnp.int32),
            pltpu.VMEM((48,), jnp.int32),
            pltpu.VMEM((48,), jnp.int32),
            pltpu.VMEM_SHARED((256,), jnp.int32),
        ],
    )(_meta_body)


def _dispatch_kernel():
    return functools.partial(
        pl.kernel,
        mesh=_MESH,
        compiler_params=_SC_PARAMS,
        out_type=jax.ShapeDtypeStruct((_SLOTS, _D), jnp.float32),
        scratch_types=[
            pltpu.VMEM((128,), jnp.int32),
            pltpu.VMEM((128,), jnp.int32),
            pltpu.VMEM((128, _D), jnp.float32),
            pltpu.SemaphoreType.DMA,
        ],
    )(_dispatch_body)


def _combine_kernel():
    return functools.partial(
        pl.kernel,
        mesh=_MESH,
        compiler_params=_SC_PARAMS,
        out_type=jax.ShapeDtypeStruct((_T, _D), jnp.float32),
        scratch_types=[
            pltpu.VMEM((128,), jnp.int32),
            pltpu.VMEM((128,), jnp.int32),
            pltpu.VMEM((128,), jnp.float32),
            pltpu.VMEM((128,), jnp.float32),
            pltpu.VMEM((32, _D), jnp.float32),
            pltpu.VMEM((32, _D), jnp.float32),
            pltpu.VMEM((32, _D), jnp.float32),
            pltpu.SemaphoreType.DMA,
        ],
    )(_combine_body)


@jax.jit
def kernel(x, Wr, br, W1, W2, W3, gamma, beta):
    Bb, Nn, Dd = x.shape
    xf = x.reshape(_T, _D)

    sel, w = pl.pallas_call(
        _router_body,
        out_shape=[jax.ShapeDtypeStruct((_T, 2), jnp.int32),
                   jax.ShapeDtypeStruct((_T, 2), jnp.float32)],
    )(xf, Wr, br)

    sel_km = sel.T.reshape(_S // 128, 128)
    w_km = w.T.reshape(_S // 128, 128)

    pos, te, nv = _meta_kernel()(sel_km)
    xs = _dispatch_kernel()(xf, pos)

    grid_spec = pltpu.PrefetchScalarGridSpec(
        num_scalar_prefetch=2,
        grid=(_NT, _NF),
        in_specs=[
            pl.BlockSpec((_TM, _D), lambda t, f, te, nv: (t, 0)),
            pl.BlockSpec((1, _D, _FT), lambda t, f, te, nv: (te[t], 0, f)),
            pl.BlockSpec((1, _D, _FT), lambda t, f, te, nv: (te[t], 0, f)),
            pl.BlockSpec((1, _FT, _D), lambda t, f, te, nv: (te[t], f, 0)),
            pl.BlockSpec((1, 1, _D), lambda t, f, te, nv: (te[t], 0, 0)),
            pl.BlockSpec((1, 1, _D), lambda t, f, te, nv: (te[t], 0, 0)),
        ],
        out_specs=pl.BlockSpec((_TM, _D), lambda t, f, te, nv: (t, 0)),
        scratch_shapes=[pltpu.VMEM((_TM, _D), jnp.float32)],
    )
    ybuf = pl.pallas_call(
        _ffn_body,
        grid_spec=grid_spec,
        out_shape=jax.ShapeDtypeStruct((_SLOTS, _D), jnp.float32),
    )(te, nv, xs, W1, W3, W2,
      gamma.reshape(_E, 1, _D), beta.reshape(_E, 1, _D))

    out = _combine_kernel()(ybuf, pos, w_km)

    return out.reshape(Bb, Nn, Dd)
